# Initial kernel scaffold; baseline (speedup 1.0000x reference)
import functools

import jax
import jax.numpy as jnp
from jax import lax
from jax.experimental import pallas as pl
from jax.experimental.pallas import tpu as pltpu
from jax.experimental.pallas import tpu_sc as plsc

E = 320000
D = 128
N = 10000
NC = 2
NS = 16
EPT = E // (NC * NS)
CHUNK = 128
NFULL = EPT // CHUNK
TAIL = EPT - NFULL * CHUNK
ZROWS = N // NS
ZCHUNK = 125


def _sc_segment_sum(M, dest):
    mesh = plsc.VectorSubcoreMesh(core_axis_name="c", subcore_axis_name="s")

    @functools.partial(
        pl.kernel,
        out_type=jax.ShapeDtypeStruct((NC, N, D), jnp.float32),
        mesh=mesh,
        scratch_types=[
            pltpu.VMEM((CHUNK, D), jnp.float32),
            pltpu.VMEM((TAIL, D), jnp.float32),
            pltpu.VMEM((CHUNK,), jnp.int32),
            pltpu.VMEM((TAIL,), jnp.int32),
            pltpu.VMEM_SHARED((N, D), jnp.float32),
        ],
    )
    def k(m_hbm, d_hbm, out_hbm, mbuf, tbuf, ibuf, itail, acc):
        c = lax.axis_index("c")
        s = lax.axis_index("s")

        zero = jnp.zeros((16,), jnp.float32)

        @pl.loop(0, CHUNK)
        def _(r):
            @pl.loop(0, D, step=16)
            def _(col):
                mbuf[r, pl.ds(col, 16)] = zero

        @pl.loop(0, ZROWS // ZCHUNK)
        def _(j):
            pltpu.sync_copy(
                mbuf.at[pl.ds(0, ZCHUNK)],
                acc.at[pl.ds(s * ZROWS + j * ZCHUNK, ZCHUNK)],
            )

        plsc.subcore_barrier()

        ebase = c * (NS * EPT) + s * EPT

        @pl.loop(0, NFULL)
        def _(j):
            off = ebase + j * CHUNK
            pltpu.sync_copy(d_hbm.at[pl.ds(off, CHUNK)], ibuf)
            pltpu.sync_copy(m_hbm.at[pl.ds(off, CHUNK)], mbuf)
            pltpu.sync_copy(mbuf, acc.at[ibuf], add=True)

        toff = ebase + NFULL * CHUNK
        pltpu.sync_copy(d_hbm.at[pl.ds(toff, TAIL)], itail)
        pltpu.sync_copy(m_hbm.at[pl.ds(toff, TAIL)], tbuf)
        pltpu.sync_copy(tbuf, acc.at[itail], add=True)

        plsc.subcore_barrier()

        @pl.loop(0, ZROWS // ZCHUNK)
        def _(j):
            row = s * ZROWS + j * ZCHUNK
            pltpu.sync_copy(
                acc.at[pl.ds(row, ZCHUNK)],
                out_hbm.at[c].at[pl.ds(row, ZCHUNK)],
            )

    return k(M, dest)


def _tc_add_kernel(a_ref, b_ref, o_ref):
    o_ref[...] = a_ref[0] + b_ref[0]


def _tc_add(partials):
    blk = 1000
    return pl.pallas_call(
        _tc_add_kernel,
        out_shape=jax.ShapeDtypeStruct((N, D), jnp.float32),
        grid=(N // blk,),
        in_specs=[
            pl.BlockSpec((1, blk, D), lambda i: (0, i, 0)),
            pl.BlockSpec((1, blk, D), lambda i: (1, i, 0)),
        ],
        out_specs=pl.BlockSpec((blk, D), lambda i: (i, 0)),
    )(partials, partials)


def kernel(M, dest, dim_size):
    dest = dest.astype(jnp.int32)
    partials = _sc_segment_sum(M, dest)
    out = _tc_add(partials)
    w = jnp.ones((E, 1), dtype=M.dtype)
    return (out, w)


# speedup vs baseline: 4.5538x; 4.5538x over previous
import functools

import jax
import jax.numpy as jnp
from jax import lax
from jax.experimental import pallas as pl
from jax.experimental.pallas import tpu as pltpu
from jax.experimental.pallas import tpu_sc as plsc

E = 320000
D = 128
N = 10000
NC = 2
NS = 16
EPT = E // (NC * NS)
CHUNK = 128
NFULL = EPT // CHUNK
TAIL = EPT - NFULL * CHUNK
ZROWS = 624
ZCHUNK = 104
ZTAIL = N - NS * ZROWS


def _sc_segment_sum(M, dest):
    mesh = plsc.VectorSubcoreMesh(core_axis_name="c", subcore_axis_name="s")

    @functools.partial(
        pl.kernel,
        out_type=jax.ShapeDtypeStruct((NC, N, D), jnp.float32),
        mesh=mesh,
        scratch_types=[
            pltpu.VMEM((CHUNK, D), jnp.float32),
            pltpu.VMEM((TAIL, D), jnp.float32),
            pltpu.VMEM((CHUNK,), jnp.int32),
            pltpu.VMEM((TAIL,), jnp.int32),
            pltpu.VMEM_SHARED((N, D), jnp.float32),
        ],
    )
    def k(m_hbm, d_hbm, out_hbm, mbuf, tbuf, ibuf, itail, acc):
        c = lax.axis_index("c")
        s = lax.axis_index("s")

        zero = jnp.zeros((16,), jnp.float32)

        @pl.loop(0, CHUNK)
        def _(r):
            @pl.loop(0, D, step=16)
            def _(col):
                mbuf[r, pl.ds(col, 16)] = zero

        @pl.loop(0, ZROWS // ZCHUNK)
        def _(j):
            pltpu.sync_copy(
                mbuf.at[pl.ds(0, ZCHUNK)],
                acc.at[pl.ds(s * ZROWS + j * ZCHUNK, ZCHUNK)],
            )

        @pl.when(s == NS - 1)
        def _():
            pltpu.sync_copy(
                mbuf.at[pl.ds(0, ZTAIL)],
                acc.at[pl.ds(NS * ZROWS, ZTAIL)],
            )

        plsc.subcore_barrier()

        ebase = c * (NS * EPT) + s * EPT

        @pl.loop(0, NFULL)
        def _(j):
            off = ebase + j * CHUNK
            pltpu.sync_copy(d_hbm.at[pl.ds(off, CHUNK)], ibuf)
            pltpu.sync_copy(m_hbm.at[pl.ds(off, CHUNK)], mbuf)
            pltpu.sync_copy(mbuf, acc.at[ibuf], add=True)

        toff = ebase + NFULL * CHUNK
        pltpu.sync_copy(d_hbm.at[pl.ds(toff, TAIL)], itail)
        pltpu.sync_copy(m_hbm.at[pl.ds(toff, TAIL)], tbuf)
        pltpu.sync_copy(tbuf, acc.at[itail], add=True)

        plsc.subcore_barrier()

        @pl.loop(0, ZROWS // ZCHUNK)
        def _(j):
            row = s * ZROWS + j * ZCHUNK
            pltpu.sync_copy(
                acc.at[pl.ds(row, ZCHUNK)],
                out_hbm.at[c].at[pl.ds(row, ZCHUNK)],
            )

        @pl.when(s == NS - 1)
        def _():
            pltpu.sync_copy(
                acc.at[pl.ds(NS * ZROWS, ZTAIL)],
                out_hbm.at[c].at[pl.ds(NS * ZROWS, ZTAIL)],
            )

    return k(M, dest)


def _tc_add_kernel(a_ref, b_ref, o_ref):
    o_ref[...] = a_ref[0] + b_ref[0]


def _tc_add(partials):
    blk = 1000
    return pl.pallas_call(
        _tc_add_kernel,
        out_shape=jax.ShapeDtypeStruct((N, D), jnp.float32),
        grid=(N // blk,),
        in_specs=[
            pl.BlockSpec((1, blk, D), lambda i: (0, i, 0)),
            pl.BlockSpec((1, blk, D), lambda i: (1, i, 0)),
        ],
        out_specs=pl.BlockSpec((blk, D), lambda i: (i, 0)),
    )(partials, partials)


def kernel(M, dest, dim_size):
    dest = dest.astype(jnp.int32)
    partials = _sc_segment_sum(M, dest)
    out = _tc_add(partials)
    w = jnp.ones((E, 1), dtype=M.dtype)
    return (out, w)


# speedup vs baseline: 6.0980x; 1.3391x over previous
import functools

import jax
import jax.numpy as jnp
from jax import lax
from jax.experimental import pallas as pl
from jax.experimental.pallas import tpu as pltpu
from jax.experimental.pallas import tpu_sc as plsc

E = 320000
D = 128
N = 10000
NC = 2
NS = 16
EPT = E // (NC * NS)
CHUNK = 128
NFULL = EPT // CHUNK
TAIL = EPT - NFULL * CHUNK
ZROWS = 624
ZCHUNK = 104
ZTAIL = N - NS * ZROWS


def _sc_segment_sum(M, dest):
    mesh = plsc.VectorSubcoreMesh(core_axis_name="c", subcore_axis_name="s")

    @functools.partial(
        pl.kernel,
        out_type=jax.ShapeDtypeStruct((NC, N, D), jnp.float32),
        mesh=mesh,
        scratch_types=[
            pltpu.VMEM((CHUNK, D), jnp.float32),
            pltpu.VMEM((CHUNK, D), jnp.float32),
            pltpu.VMEM((CHUNK,), jnp.int32),
            pltpu.VMEM((CHUNK,), jnp.int32),
            pltpu.VMEM((TAIL, D), jnp.float32),
            pltpu.VMEM((TAIL,), jnp.int32),
            pltpu.VMEM_SHARED((N, D), jnp.float32),
            pltpu.SemaphoreType.DMA,
            pltpu.SemaphoreType.DMA,
            pltpu.SemaphoreType.DMA,
            pltpu.SemaphoreType.DMA,
        ],
    )
    def k(m_hbm, d_hbm, out_hbm, mb0, mb1, ib0, ib1, tbuf, itail, acc,
          ms0, ms1, is0, is1):
        c = lax.axis_index("c")
        s = lax.axis_index("s")

        zero = jnp.zeros((16,), jnp.float32)

        @pl.loop(0, CHUNK)
        def _(r):
            @pl.loop(0, D, step=16)
            def _(col):
                mb0[r, pl.ds(col, 16)] = zero

        @pl.loop(0, ZROWS // ZCHUNK)
        def _(j):
            pltpu.sync_copy(
                mb0.at[pl.ds(0, ZCHUNK)],
                acc.at[pl.ds(s * ZROWS + j * ZCHUNK, ZCHUNK)],
            )

        @pl.when(s == NS - 1)
        def _():
            pltpu.sync_copy(
                mb0.at[pl.ds(0, ZTAIL)],
                acc.at[pl.ds(NS * ZROWS, ZTAIL)],
            )

        plsc.subcore_barrier()

        ebase = c * (NS * EPT) + s * EPT

        def start_in(j, mb, ib, msem, isem):
            off = ebase + j * CHUNK
            pltpu.async_copy(d_hbm.at[pl.ds(off, CHUNK)], ib, isem)
            pltpu.async_copy(m_hbm.at[pl.ds(off, CHUNK)], mb, msem)

        def wait_in(mb, ib, msem, isem):
            pltpu.make_async_copy(d_hbm.at[pl.ds(0, CHUNK)], ib, isem).wait()
            pltpu.make_async_copy(m_hbm.at[pl.ds(0, CHUNK)], mb, msem).wait()

        start_in(0, mb0, ib0, ms0, is0)
        start_in(1, mb1, ib1, ms1, is1)

        @pl.loop(0, NFULL // 2)
        def _(it):
            j0 = it * 2

            wait_in(mb0, ib0, ms0, is0)
            h0 = pltpu.async_copy(mb0, acc.at[ib0], ms0, add=True)

            wait_in(mb1, ib1, ms1, is1)
            h1 = pltpu.async_copy(mb1, acc.at[ib1], ms1, add=True)

            h0.wait()

            @pl.when(j0 + 2 < NFULL)
            def _():
                start_in(j0 + 2, mb0, ib0, ms0, is0)

            h1.wait()

            @pl.when(j0 + 3 < NFULL)
            def _():
                start_in(j0 + 3, mb1, ib1, ms1, is1)

        toff = ebase + NFULL * CHUNK
        pltpu.sync_copy(d_hbm.at[pl.ds(toff, TAIL)], itail)
        pltpu.sync_copy(m_hbm.at[pl.ds(toff, TAIL)], tbuf)
        pltpu.sync_copy(tbuf, acc.at[itail], add=True)

        plsc.subcore_barrier()

        @pl.loop(0, ZROWS // ZCHUNK)
        def _(j):
            row = s * ZROWS + j * ZCHUNK
            pltpu.sync_copy(
                acc.at[pl.ds(row, ZCHUNK)],
                out_hbm.at[c].at[pl.ds(row, ZCHUNK)],
            )

        @pl.when(s == NS - 1)
        def _():
            pltpu.sync_copy(
                acc.at[pl.ds(NS * ZROWS, ZTAIL)],
                out_hbm.at[c].at[pl.ds(NS * ZROWS, ZTAIL)],
            )

    return k(M, dest)


def _tc_add_kernel(a_ref, b_ref, o_ref):
    o_ref[...] = a_ref[0] + b_ref[0]


def _tc_add(partials):
    blk = 1000
    return pl.pallas_call(
        _tc_add_kernel,
        out_shape=jax.ShapeDtypeStruct((N, D), jnp.float32),
        grid=(N // blk,),
        in_specs=[
            pl.BlockSpec((1, blk, D), lambda i: (0, i, 0)),
            pl.BlockSpec((1, blk, D), lambda i: (1, i, 0)),
        ],
        out_specs=pl.BlockSpec((blk, D), lambda i: (i, 0)),
    )(partials, partials)


def kernel(M, dest, dim_size):
    dest = dest.astype(jnp.int32)
    partials = _sc_segment_sum(M, dest)
    out = _tc_add(partials)
    w = jnp.ones((E, 1), dtype=M.dtype)
    return (out, w)


# speedup vs baseline: 7.1745x; 1.1765x over previous
import functools

import jax
import jax.numpy as jnp
from jax import lax
from jax.experimental import pallas as pl
from jax.experimental.pallas import tpu as pltpu
from jax.experimental.pallas import tpu_sc as plsc

E = 320000
D = 128
N = 10000
NC = 2
NS = 16
EPT = E // (NC * NS)
CHUNK = 128
NBUF = 3
NFULL = EPT // CHUNK
TAIL = EPT - NFULL * CHUNK
ZROWS = 624
ZCHUNK = 104
ZTAIL = N - NS * ZROWS


def _sc_segment_sum(M, dest):
    mesh = plsc.VectorSubcoreMesh(core_axis_name="c", subcore_axis_name="s")

    @functools.partial(
        pl.kernel,
        out_type=jax.ShapeDtypeStruct((NC, N, D), jnp.float32),
        mesh=mesh,
        scratch_types=[
            pltpu.VMEM((CHUNK, D), jnp.float32),
            pltpu.VMEM((CHUNK, D), jnp.float32),
            pltpu.VMEM((CHUNK, D), jnp.float32),
            pltpu.VMEM((CHUNK,), jnp.int32),
            pltpu.VMEM((CHUNK,), jnp.int32),
            pltpu.VMEM((CHUNK,), jnp.int32),
            pltpu.VMEM((TAIL,), jnp.int32),
            pltpu.VMEM_SHARED((N, D), jnp.float32),
            pltpu.SemaphoreType.DMA,
            pltpu.SemaphoreType.DMA,
            pltpu.SemaphoreType.DMA,
            pltpu.SemaphoreType.DMA,
            pltpu.SemaphoreType.DMA,
            pltpu.SemaphoreType.DMA,
        ],
    )
    def k(m_hbm, d_hbm, out_hbm, mb0, mb1, mb2, ib0, ib1, ib2, itail,
          acc, ms0, ms1, ms2, is0, is1, is2):
        c = lax.axis_index("c")
        s = lax.axis_index("s")
        mbs = (mb0, mb1, mb2)
        ibs = (ib0, ib1, ib2)
        msems = (ms0, ms1, ms2)
        isems = (is0, is1, is2)

        zero = jnp.zeros((16,), jnp.float32)

        @pl.loop(0, ZCHUNK)
        def _(r):
            @pl.loop(0, D, step=16)
            def _(col):
                mb0[r, pl.ds(col, 16)] = zero

        @pl.loop(0, ZROWS // ZCHUNK)
        def _(j):
            pltpu.sync_copy(
                mb0.at[pl.ds(0, ZCHUNK)],
                acc.at[pl.ds(s * ZROWS + j * ZCHUNK, ZCHUNK)],
            )

        @pl.when(s == NS - 1)
        def _():
            pltpu.sync_copy(
                mb0.at[pl.ds(0, ZTAIL)],
                acc.at[pl.ds(NS * ZROWS, ZTAIL)],
            )

        plsc.subcore_barrier()

        ebase = c * (NS * EPT) + s * EPT

        def start_in(j, b):
            off = ebase + j * CHUNK
            pltpu.async_copy(d_hbm.at[pl.ds(off, CHUNK)], ibs[b], isems[b])
            pltpu.async_copy(m_hbm.at[pl.ds(off, CHUNK)], mbs[b], msems[b])

        def wait_in(b):
            pltpu.make_async_copy(
                d_hbm.at[pl.ds(0, CHUNK)], ibs[b], isems[b]).wait()
            pltpu.make_async_copy(
                m_hbm.at[pl.ds(0, CHUNK)], mbs[b], msems[b]).wait()

        for b in range(NBUF):
            start_in(b, b)

        @pl.loop(0, NFULL // NBUF)
        def _(it):
            j0 = it * NBUF
            hs = []
            for b in range(NBUF):
                wait_in(b)
                hs.append(pltpu.async_copy(
                    mbs[b], acc.at[ibs[b]], msems[b], add=True))
            for b in range(NBUF):
                hs[b].wait()

                @pl.when(j0 + b + NBUF < NFULL)
                def _(b=b):
                    start_in(j0 + b + NBUF, b)

        toff = ebase + NFULL * CHUNK
        pltpu.sync_copy(d_hbm.at[pl.ds(toff, TAIL)], itail)
        pltpu.sync_copy(m_hbm.at[pl.ds(toff, TAIL)], mb0.at[pl.ds(0, TAIL)])
        pltpu.sync_copy(mb0.at[pl.ds(0, TAIL)], acc.at[itail], add=True)

        plsc.subcore_barrier()

        @pl.loop(0, ZROWS // ZCHUNK)
        def _(j):
            row = s * ZROWS + j * ZCHUNK
            pltpu.sync_copy(
                acc.at[pl.ds(row, ZCHUNK)],
                out_hbm.at[c].at[pl.ds(row, ZCHUNK)],
            )

        @pl.when(s == NS - 1)
        def _():
            pltpu.sync_copy(
                acc.at[pl.ds(NS * ZROWS, ZTAIL)],
                out_hbm.at[c].at[pl.ds(NS * ZROWS, ZTAIL)],
            )

    return k(M, dest)


def _tc_add_kernel(a_ref, b_ref, o_ref):
    o_ref[...] = a_ref[0] + b_ref[0]


def _tc_add(partials):
    blk = 1000
    return pl.pallas_call(
        _tc_add_kernel,
        out_shape=jax.ShapeDtypeStruct((N, D), jnp.float32),
        grid=(N // blk,),
        in_specs=[
            pl.BlockSpec((1, blk, D), lambda i: (0, i, 0)),
            pl.BlockSpec((1, blk, D), lambda i: (1, i, 0)),
        ],
        out_specs=pl.BlockSpec((blk, D), lambda i: (i, 0)),
    )(partials, partials)


def kernel(M, dest, dim_size):
    partials = _sc_segment_sum(M, dest.astype(jnp.int32))
    out = _tc_add(partials)
    w = jnp.ones((E, 1), dtype=M.dtype)
    return (out, w)
